# Initial kernel scaffold; baseline (speedup 1.0000x reference)
import jax
import jax.numpy as jnp
from jax.experimental import pallas as pl


def kernel(x, img_shape, conv1_w, conv1_b, score_w, score_b, offset_w, offset_b):
    raise NotImplementedError("write your pallas kernel here")


# speedup vs baseline: 1.3240x; 1.3240x over previous
import functools

import jax
import jax.numpy as jnp
from jax.experimental import pallas as pl
from jax.experimental.pallas import tpu as pltpu

N, C, H, W = 4, 256, 64, 64
P = H * W
PPAD = P + 2 * W
A = 9
AP = 16
BR = 512
R = P // BR


def _rpn_kernel(x_ref, w_ref, b_ref, dw_ref, db_ref, o_ref, xl_ref, xr_ref):
    r = pl.program_id(1)

    @pl.when(r == 0)
    def _build_shifted():
        xc = x_ref[0]
        col = jax.lax.broadcasted_iota(jnp.int32, (PPAD, C), 0) % W
        xl = pltpu.roll(xc, PPAD - 1, 0)
        xl_ref[...] = jnp.where(col != (W - 1), xl, 0.0)
        xr = pltpu.roll(xc, 1, 0)
        xr_ref[...] = jnp.where(col != 0, xr, 0.0)

    acc = jnp.zeros((BR, C), dtype=jnp.float32)
    base = W + r * BR
    for dy in (-1, 0, 1):
        start = base + dy * W
        for dx, src in ((-1, xr_ref), (0, x_ref), (1, xl_ref)):
            k = (dy + 1) * 3 + (dx + 1)
            if src is x_ref:
                blk = src[0, pl.ds(start, BR), :]
            else:
                blk = src[pl.ds(start, BR), :]
            acc += jnp.dot(blk, w_ref[k], preferred_element_type=jnp.float32)
    t = jax.nn.relu(acc + b_ref[0])
    s = jnp.dot(t, dw_ref[...], preferred_element_type=jnp.float32) + db_ref[0]
    o_ref[0] = jax.nn.sigmoid(s)


@functools.partial(jax.jit, static_argnames=())
def kernel(x, img_shape, conv1_w, conv1_b, score_w, score_b, offset_w, offset_b):
    n = x.shape[0]
    xt = jnp.transpose(x, (0, 2, 3, 1)).reshape(n, P, C)
    xt = jnp.pad(xt, ((0, 0), (W, W), (0, 0)))
    wr = jnp.transpose(conv1_w, (2, 3, 1, 0)).reshape(9, C, C)
    b2 = conv1_b.reshape(1, C)
    sw = score_w[:, :, 0, 0]
    dw = (sw[1::2] - sw[0::2]).T
    dw = jnp.pad(dw, ((0, 0), (0, AP - A)))
    db = jnp.pad(score_b[1::2] - score_b[0::2], (0, AP - A)).reshape(1, AP)

    fg = pl.pallas_call(
        _rpn_kernel,
        grid=(n, R),
        in_specs=[
            pl.BlockSpec((1, PPAD, C), lambda i, r: (i, 0, 0)),
            pl.BlockSpec((9, C, C), lambda i, r: (0, 0, 0)),
            pl.BlockSpec((1, C), lambda i, r: (0, 0)),
            pl.BlockSpec((C, AP), lambda i, r: (0, 0)),
            pl.BlockSpec((1, AP), lambda i, r: (0, 0)),
        ],
        out_specs=pl.BlockSpec((1, BR, AP), lambda i, r: (i, r, 0)),
        out_shape=jax.ShapeDtypeStruct((n, P, AP), jnp.float32),
        scratch_shapes=[
            pltpu.VMEM((PPAD, C), jnp.float32),
            pltpu.VMEM((PPAD, C), jnp.float32),
        ],
    )(xt, wr, b2, dw, db)

    return fg[:, :, :A].reshape(n, P * A // 2, 2)
